# Initial kernel scaffold; baseline (speedup 1.0000x reference)
import jax
import jax.numpy as jnp
from jax.experimental import pallas as pl


def kernel(xyz, new_xyz, features, W1, b1, g1, bt1, W2, b2, g2, bt2):
    raise NotImplementedError("write your pallas kernel here")


# speedup vs baseline: 12.4401x; 12.4401x over previous
import functools

import jax
import jax.numpy as jnp
from jax import lax
from jax.experimental import pallas as pl
from jax.experimental.pallas import tpu as pltpu
from jax.experimental.pallas import tpu_sc as plsc

_B, _N, _M, _C, _NS = 4, 4096, 1024, 64, 32
_R2 = 0.12 ** 2
_H1, _H2 = 64, 128
_P = 128
_BT = _B * _M * _NS
_MB = 256
_NCH = _N // 128
_QB = 64
_PB = _QB * _NS


def _bq_body(xyzT_ref, nq_ref, out_ref):
    b = pl.program_id(0)
    t = xyzT_ref[...]
    q = nq_ref[...][0]
    px, py, pz = t[0, 0:1, :], t[0, 1:2, :], t[0, 2:3, :]
    dx = q[:, 0:1] - px
    dy = q[:, 1:2] - py
    dz = q[:, 2:3] - pz
    d2 = dx * dx + dy * dy + dz * dz
    m = (d2 <= _R2).astype(jnp.float32)

    ri = lax.broadcasted_iota(jnp.int32, (128, 128), 0)
    cj = lax.broadcasted_iota(jnp.int32, (128, 128), 1)
    lt = (ri <= cj).astype(jnp.float32)

    base = jnp.zeros((_MB, 1), jnp.float32)
    ranks = []
    for c in range(_NCH):
        mc = m[:, c * 128:(c + 1) * 128]
        loc = jnp.dot(mc, lt, preferred_element_type=jnp.float32)
        ranks.append(loc + base)
        base = base + loc[:, 127:128]
    rank = jnp.concatenate(ranks, axis=1)

    cols = []
    for k in range(_NS):
        cmp = (rank <= jnp.float32(k)).astype(jnp.float32)
        cols.append(jnp.sum(cmp, axis=1, keepdims=True))
    cnt = jnp.concatenate(cols, axis=1)
    first = cnt[:, 0:1]
    sent = jnp.float32(_N)
    cnt = jnp.where(cnt == sent, jnp.broadcast_to(first, cnt.shape), cnt)
    cnt = jnp.where(cnt == sent, jnp.float32(0), cnt)
    out_ref[...] = (cnt.astype(jnp.int32) + b * _N)[None]


def _ball_query(xyzT, new_xyz):
    return pl.pallas_call(
        _bq_body,
        grid=(_B, _M // _MB),
        in_specs=[
            pl.BlockSpec((1, 3, _N), lambda b, i: (b, 0, 0)),
            pl.BlockSpec((1, _MB, 3), lambda b, i: (b, i, 0)),
        ],
        out_specs=pl.BlockSpec((1, _MB, _NS), lambda b, i: (b, i, 0)),
        out_shape=jax.ShapeDtypeStruct((_B, _M, _NS), jnp.int32),
    )(xyzT, new_xyz)


_NW = 32
_PER_W = _BT // _NW
_CH = 256


@functools.lru_cache(maxsize=1)
def _make_gather():
    @functools.partial(
        pl.kernel,
        out_type=jax.ShapeDtypeStruct((_BT, _P), jnp.float32),
        mesh=plsc.VectorSubcoreMesh(core_axis_name="c", subcore_axis_name="s"),
        scratch_types=[
            pltpu.VMEM((_CH,), jnp.int32),
            pltpu.VMEM((_CH, _P), jnp.float32),
            pltpu.SemaphoreType.DMA,
        ],
    )
    def gather(table_hbm, idx_hbm, out_hbm, idx_v, rows_v, sem):
        wid = lax.axis_index("s") * 2 + lax.axis_index("c")
        base = wid * _PER_W
        for ch in range(_PER_W // _CH):
            off = base + ch * _CH
            pltpu.sync_copy(idx_hbm.at[pl.ds(off, _CH)], idx_v)
            pltpu.async_copy(table_hbm.at[idx_v], rows_v, sem).wait()
            pltpu.sync_copy(rows_v, out_hbm.at[pl.ds(off, _CH)])

    return gather


def _gather_rows(table, idx):
    return _make_gather()(table, idx)


def _k1_body(xg_ref, nq_ref, w1_ref, w1x_ref, b1_ref, y1_ref, s_ref, ss_ref):
    i = pl.program_id(0)
    xg = xg_ref[...]
    q = nq_ref[...]
    corr = jnp.dot(q, w1x_ref[...], preferred_element_type=jnp.float32)
    ri = lax.broadcasted_iota(jnp.int32, (_PB, _QB), 0)
    cj = lax.broadcasted_iota(jnp.int32, (_PB, _QB), 1)
    rep = (ri // _NS == cj).astype(jnp.float32)
    corr_rep = jnp.dot(rep, corr, preferred_element_type=jnp.float32)
    y1 = (jnp.dot(xg, w1_ref[...], preferred_element_type=jnp.float32)
          + b1_ref[...] - corr_rep)
    y1_ref[...] = y1
    ps = jnp.sum(y1, axis=0, keepdims=True)
    pss = jnp.sum(y1 * y1, axis=0, keepdims=True)

    @pl.when(i == 0)
    def _init():
        s_ref[...] = ps
        ss_ref[...] = pss

    @pl.when(i > 0)
    def _acc():
        s_ref[...] += ps
        ss_ref[...] += pss


def _k2_body(y1_ref, s1_ref, ss1_ref, g1_ref, bt1_ref, w2_ref, b2_ref,
             y2_ref, s_ref, ss_ref):
    i = pl.program_id(0)
    inv_n = jnp.float32(1.0 / _BT)
    m1 = s1_ref[...] * inv_n
    var1 = ss1_ref[...] * inv_n - m1 * m1
    sc = g1_ref[...] / jnp.sqrt(var1 + 1e-5)
    z = jnp.maximum((y1_ref[...] - m1) * sc + bt1_ref[...], 0.0)
    y2 = (jnp.dot(z, w2_ref[...], preferred_element_type=jnp.float32)
          + b2_ref[...])
    y2_ref[...] = y2
    ps = jnp.sum(y2, axis=0, keepdims=True)
    pss = jnp.sum(y2 * y2, axis=0, keepdims=True)

    @pl.when(i == 0)
    def _init():
        s_ref[...] = ps
        ss_ref[...] = pss

    @pl.when(i > 0)
    def _acc():
        s_ref[...] += ps
        ss_ref[...] += pss


def _k3_body(y2_ref, s2_ref, ss2_ref, g2_ref, bt2_ref, out_ref):
    inv_n = jnp.float32(1.0 / _BT)
    m2 = s2_ref[...] * inv_n
    var2 = ss2_ref[...] * inv_n - m2 * m2
    sc = g2_ref[...] / jnp.sqrt(var2 + 1e-5)
    y = y2_ref[...]
    z = jnp.maximum((y - m2) * sc + bt2_ref[...], 0.0)
    red = z[:, 0, :]
    for k in range(1, _NS):
        red = jnp.maximum(red, z[:, k, :])
    out_ref[...] = red


def _whole(shape):
    return pl.BlockSpec(shape, lambda i: tuple(0 for _ in shape))


def kernel(xyz, new_xyz, features, W1, b1, g1, bt1, W2, b2, g2, bt2):
    xyzT = jnp.transpose(xyz, (0, 2, 1))
    ftT = jnp.transpose(features, (0, 2, 1))
    pad = jnp.zeros((_B, _N, _P - 3 - _C), jnp.float32)
    table = jnp.concatenate([xyz, ftT, pad], axis=-1).reshape(_B * _N, _P)

    idx = _ball_query(xyzT, new_xyz)
    xg = _gather_rows(table, idx.reshape(_BT))

    nq_flat = new_xyz.reshape(_B * _M, 3)
    w1p = jnp.pad(W1, ((0, 0), (0, _P - 3 - _C))).T
    w1x = W1[:, :3].T

    nblk = _BT // _PB
    y1, s1, ss1 = pl.pallas_call(
        _k1_body,
        grid=(nblk,),
        in_specs=[
            pl.BlockSpec((_PB, _P), lambda i: (i, 0)),
            pl.BlockSpec((_QB, 3), lambda i: (i, 0)),
            _whole((_P, _H1)),
            _whole((3, _H1)),
            _whole((1, _H1)),
        ],
        out_specs=[
            pl.BlockSpec((_PB, _H1), lambda i: (i, 0)),
            _whole((1, _H1)),
            _whole((1, _H1)),
        ],
        out_shape=[
            jax.ShapeDtypeStruct((_BT, _H1), jnp.float32),
            jax.ShapeDtypeStruct((1, _H1), jnp.float32),
            jax.ShapeDtypeStruct((1, _H1), jnp.float32),
        ],
    )(xg, nq_flat, w1p, w1x, b1[None])

    y2, s2, ss2 = pl.pallas_call(
        _k2_body,
        grid=(nblk,),
        in_specs=[
            pl.BlockSpec((_PB, _H1), lambda i: (i, 0)),
            _whole((1, _H1)),
            _whole((1, _H1)),
            _whole((1, _H1)),
            _whole((1, _H1)),
            _whole((_H1, _H2)),
            _whole((1, _H2)),
        ],
        out_specs=[
            pl.BlockSpec((_PB, _H2), lambda i: (i, 0)),
            _whole((1, _H2)),
            _whole((1, _H2)),
        ],
        out_shape=[
            jax.ShapeDtypeStruct((_BT, _H2), jnp.float32),
            jax.ShapeDtypeStruct((1, _H2), jnp.float32),
            jax.ShapeDtypeStruct((1, _H2), jnp.float32),
        ],
    )(y1, s1, ss1, g1[None], bt1[None], W2.T, b2[None])

    pooled = pl.pallas_call(
        _k3_body,
        grid=(_B * _M // _QB,),
        in_specs=[
            pl.BlockSpec((_QB, _NS, _H2), lambda i: (i, 0, 0)),
            _whole((1, _H2)),
            _whole((1, _H2)),
            _whole((1, _H2)),
            _whole((1, _H2)),
        ],
        out_specs=pl.BlockSpec((_QB, _H2), lambda i: (i, 0)),
        out_shape=jax.ShapeDtypeStruct((_B * _M, _H2), jnp.float32),
    )(y2.reshape(_B * _M, _NS, _H2), s2, ss2, g2[None], bt2[None])

    out = pooled.reshape(_B, _M, _H2).transpose(0, 2, 1)
    return (new_xyz, out)


# speedup vs baseline: 13.0644x; 1.0502x over previous
import functools

import jax
import jax.numpy as jnp
from jax import lax
from jax.experimental import pallas as pl
from jax.experimental.pallas import tpu as pltpu
from jax.experimental.pallas import tpu_sc as plsc

_B, _N, _M, _C, _NS = 4, 4096, 1024, 64, 32
_R2 = 0.12 ** 2
_H1, _H2 = 64, 128
_P = 128
_BT = _B * _M * _NS
_MB = 256
_NCH = _N // 128
_QB = 64
_PB = _QB * _NS


def _bq_body(xyzT_ref, nq_ref, out_ref):
    b = pl.program_id(0)
    t = xyzT_ref[...]
    q = nq_ref[...][0]
    p3 = t[0]
    pn = (p3[0:1, :] * p3[0:1, :] + p3[1:2, :] * p3[1:2, :]
          + p3[2:3, :] * p3[2:3, :])
    qn = (q[:, 0:1] * q[:, 0:1] + q[:, 1:2] * q[:, 1:2]
          + q[:, 2:3] * q[:, 2:3])
    lhs = jnp.concatenate([q, qn, jnp.ones((_MB, 1), jnp.float32)], axis=1)
    rhs = jnp.concatenate([-2.0 * p3, jnp.ones((1, _N), jnp.float32), pn],
                          axis=0)
    d2 = jnp.dot(lhs, rhs, precision=lax.Precision.HIGHEST,
                 preferred_element_type=jnp.float32)
    m = (d2 <= _R2).astype(jnp.bfloat16)

    ri = lax.broadcasted_iota(jnp.int32, (128, 128), 0)
    cj = lax.broadcasted_iota(jnp.int32, (128, 128), 1)
    lt = (ri <= cj).astype(jnp.bfloat16)

    base = jnp.zeros((_MB, 1), jnp.float32)
    ranks = []
    for c in range(_NCH):
        mc = m[:, c * 128:(c + 1) * 128]
        loc = jnp.dot(mc, lt, preferred_element_type=jnp.float32)
        ranks.append(jnp.minimum(loc + base, 34.0))
        base = base + loc[:, 127:128]
    rank = jnp.concatenate(ranks, axis=1)

    s_prev = jnp.zeros((_MB, 1), jnp.float32)
    cols = []
    for k in range(_NS):
        s_cur = jnp.sum(jnp.minimum(rank, jnp.float32(k + 1)), axis=1,
                        keepdims=True)
        cols.append(jnp.float32(_N) - s_cur + s_prev)
        s_prev = s_cur
    cnt = jnp.concatenate(cols, axis=1)
    first = cnt[:, 0:1]
    sent = jnp.float32(_N)
    cnt = jnp.where(cnt == sent, jnp.broadcast_to(first, cnt.shape), cnt)
    cnt = jnp.where(cnt == sent, jnp.float32(0), cnt)
    out_ref[...] = (cnt.astype(jnp.int32) + b * _N)[None]


def _ball_query(xyzT, new_xyz):
    return pl.pallas_call(
        _bq_body,
        grid=(_B, _M // _MB),
        in_specs=[
            pl.BlockSpec((1, 3, _N), lambda b, i: (b, 0, 0)),
            pl.BlockSpec((1, _MB, 3), lambda b, i: (b, i, 0)),
        ],
        out_specs=pl.BlockSpec((1, _MB, _NS), lambda b, i: (b, i, 0)),
        out_shape=jax.ShapeDtypeStruct((_B, _M, _NS), jnp.int32),
    )(xyzT, new_xyz)


_NW = 32
_PER_W = _BT // _NW
_CH = 256


@functools.lru_cache(maxsize=1)
def _make_gather():
    @functools.partial(
        pl.kernel,
        out_type=jax.ShapeDtypeStruct((_BT, _P), jnp.float32),
        mesh=plsc.VectorSubcoreMesh(core_axis_name="c", subcore_axis_name="s"),
        scratch_types=[
            pltpu.VMEM((_CH,), jnp.int32),
            pltpu.VMEM((_CH, _P), jnp.float32),
            pltpu.SemaphoreType.DMA,
        ],
    )
    def gather(table_hbm, idx_hbm, out_hbm, idx_v, rows_v, sem):
        wid = lax.axis_index("s") * 2 + lax.axis_index("c")
        base = wid * _PER_W
        for ch in range(_PER_W // _CH):
            off = base + ch * _CH
            pltpu.sync_copy(idx_hbm.at[pl.ds(off, _CH)], idx_v)
            pltpu.async_copy(table_hbm.at[idx_v], rows_v, sem).wait()
            pltpu.sync_copy(rows_v, out_hbm.at[pl.ds(off, _CH)])

    return gather


def _gather_rows(table, idx):
    return _make_gather()(table, idx)


def _k1_body(xg_ref, nq_ref, w1_ref, w1x_ref, b1_ref, y1_ref, s_ref, ss_ref):
    i = pl.program_id(0)
    xg = xg_ref[...]
    q = nq_ref[...]
    corr = jnp.dot(q, w1x_ref[...], preferred_element_type=jnp.float32)
    ri = lax.broadcasted_iota(jnp.int32, (_PB, _QB), 0)
    cj = lax.broadcasted_iota(jnp.int32, (_PB, _QB), 1)
    rep = (ri // _NS == cj).astype(jnp.float32)
    corr_rep = jnp.dot(rep, corr, preferred_element_type=jnp.float32)
    y1 = (jnp.dot(xg, w1_ref[...], preferred_element_type=jnp.float32)
          + b1_ref[...] - corr_rep)
    y1_ref[...] = y1
    ps = jnp.sum(y1, axis=0, keepdims=True)
    pss = jnp.sum(y1 * y1, axis=0, keepdims=True)

    @pl.when(i == 0)
    def _init():
        s_ref[...] = ps
        ss_ref[...] = pss

    @pl.when(i > 0)
    def _acc():
        s_ref[...] += ps
        ss_ref[...] += pss


def _k2_body(y1_ref, s1_ref, ss1_ref, g1_ref, bt1_ref, w2_ref, b2_ref,
             y2_ref, s_ref, ss_ref):
    i = pl.program_id(0)
    inv_n = jnp.float32(1.0 / _BT)
    m1 = s1_ref[...] * inv_n
    var1 = ss1_ref[...] * inv_n - m1 * m1
    sc = g1_ref[...] / jnp.sqrt(var1 + 1e-5)
    z = jnp.maximum((y1_ref[...] - m1) * sc + bt1_ref[...], 0.0)
    y2 = (jnp.dot(z, w2_ref[...], preferred_element_type=jnp.float32)
          + b2_ref[...])
    y2_ref[...] = y2
    ps = jnp.sum(y2, axis=0, keepdims=True)
    pss = jnp.sum(y2 * y2, axis=0, keepdims=True)

    @pl.when(i == 0)
    def _init():
        s_ref[...] = ps
        ss_ref[...] = pss

    @pl.when(i > 0)
    def _acc():
        s_ref[...] += ps
        ss_ref[...] += pss


def _k3_body(y2_ref, s2_ref, ss2_ref, g2_ref, bt2_ref, out_ref):
    inv_n = jnp.float32(1.0 / _BT)
    m2 = s2_ref[...] * inv_n
    var2 = ss2_ref[...] * inv_n - m2 * m2
    sc = g2_ref[...] / jnp.sqrt(var2 + 1e-5)
    y = y2_ref[...]
    z = jnp.maximum((y - m2) * sc + bt2_ref[...], 0.0)
    red = z[:, 0, :]
    for k in range(1, _NS):
        red = jnp.maximum(red, z[:, k, :])
    out_ref[...] = red


def _whole(shape):
    return pl.BlockSpec(shape, lambda i: tuple(0 for _ in shape))


def kernel(xyz, new_xyz, features, W1, b1, g1, bt1, W2, b2, g2, bt2):
    xyzT = jnp.transpose(xyz, (0, 2, 1))
    ftT = jnp.transpose(features, (0, 2, 1))
    pad = jnp.zeros((_B, _N, _P - 3 - _C), jnp.float32)
    table = jnp.concatenate([xyz, ftT, pad], axis=-1).reshape(_B * _N, _P)

    idx = _ball_query(xyzT, new_xyz)
    xg = _gather_rows(table, idx.reshape(_BT))

    nq_flat = new_xyz.reshape(_B * _M, 3)
    w1p = jnp.pad(W1, ((0, 0), (0, _P - 3 - _C))).T
    w1x = W1[:, :3].T

    nblk = _BT // _PB
    y1, s1, ss1 = pl.pallas_call(
        _k1_body,
        grid=(nblk,),
        in_specs=[
            pl.BlockSpec((_PB, _P), lambda i: (i, 0)),
            pl.BlockSpec((_QB, 3), lambda i: (i, 0)),
            _whole((_P, _H1)),
            _whole((3, _H1)),
            _whole((1, _H1)),
        ],
        out_specs=[
            pl.BlockSpec((_PB, _H1), lambda i: (i, 0)),
            _whole((1, _H1)),
            _whole((1, _H1)),
        ],
        out_shape=[
            jax.ShapeDtypeStruct((_BT, _H1), jnp.float32),
            jax.ShapeDtypeStruct((1, _H1), jnp.float32),
            jax.ShapeDtypeStruct((1, _H1), jnp.float32),
        ],
    )(xg, nq_flat, w1p, w1x, b1[None])

    y2, s2, ss2 = pl.pallas_call(
        _k2_body,
        grid=(nblk,),
        in_specs=[
            pl.BlockSpec((_PB, _H1), lambda i: (i, 0)),
            _whole((1, _H1)),
            _whole((1, _H1)),
            _whole((1, _H1)),
            _whole((1, _H1)),
            _whole((_H1, _H2)),
            _whole((1, _H2)),
        ],
        out_specs=[
            pl.BlockSpec((_PB, _H2), lambda i: (i, 0)),
            _whole((1, _H2)),
            _whole((1, _H2)),
        ],
        out_shape=[
            jax.ShapeDtypeStruct((_BT, _H2), jnp.float32),
            jax.ShapeDtypeStruct((1, _H2), jnp.float32),
            jax.ShapeDtypeStruct((1, _H2), jnp.float32),
        ],
    )(y1, s1, ss1, g1[None], bt1[None], W2.T, b2[None])

    pooled = pl.pallas_call(
        _k3_body,
        grid=(_B * _M // _QB,),
        in_specs=[
            pl.BlockSpec((_QB, _NS, _H2), lambda i: (i, 0, 0)),
            _whole((1, _H2)),
            _whole((1, _H2)),
            _whole((1, _H2)),
            _whole((1, _H2)),
        ],
        out_specs=pl.BlockSpec((_QB, _H2), lambda i: (i, 0)),
        out_shape=jax.ShapeDtypeStruct((_B * _M, _H2), jnp.float32),
    )(y2.reshape(_B * _M, _NS, _H2), s2, ss2, g2[None], bt2[None])

    out = pooled.reshape(_B, _M, _H2).transpose(0, 2, 1)
    return (new_xyz, out)


# speedup vs baseline: 15.7385x; 1.2047x over previous
import functools

import jax
import jax.numpy as jnp
import numpy as np
from jax import lax
from jax.experimental import pallas as pl
from jax.experimental.pallas import tpu as pltpu
from jax.experimental.pallas import tpu_sc as plsc

_B, _N, _M, _C, _NS = 4, 4096, 1024, 64, 32
_R2 = 0.12 ** 2
_H1, _H2 = 64, 128
_P = 128
_BT = _B * _M * _NS
_MB = 256
_NCH = _N // 128
_QB = 64
_PB = _QB * _NS


_NWORD = _N // 16


def _pack_body(xyzT_ref, nq_ref, pk_ref, out_ref):
    t = xyzT_ref[...]
    q = nq_ref[...][0]
    p3 = t[0]
    pn = (p3[0:1, :] * p3[0:1, :] + p3[1:2, :] * p3[1:2, :]
          + p3[2:3, :] * p3[2:3, :])
    qn = (q[:, 0:1] * q[:, 0:1] + q[:, 1:2] * q[:, 1:2]
          + q[:, 2:3] * q[:, 2:3])
    lhs = jnp.concatenate([q, qn, jnp.ones((_MB, 1), jnp.float32)], axis=1)
    rhs = jnp.concatenate([-2.0 * p3, jnp.ones((1, _N), jnp.float32), pn],
                          axis=0)
    d2 = jnp.dot(lhs, rhs, precision=lax.Precision.HIGHEST,
                 preferred_element_type=jnp.float32)
    m = (d2 <= _R2).astype(jnp.bfloat16)
    words = jnp.dot(m, pk_ref[...], preferred_element_type=jnp.float32)
    out_ref[...] = words.astype(jnp.int32)


def _pack_words(xyzT, new_xyz, pk):
    return pl.pallas_call(
        _pack_body,
        grid=(_B, _M // _MB),
        in_specs=[
            pl.BlockSpec((1, 3, _N), lambda b, i: (b, 0, 0)),
            pl.BlockSpec((1, _MB, 3), lambda b, i: (b, i, 0)),
            pl.BlockSpec((_N, _NWORD), lambda b, i: (0, 0)),
        ],
        out_specs=pl.BlockSpec((_MB, _NWORD),
                               lambda b, i: (b * (_M // _MB) + i, 0)),
        out_shape=jax.ShapeDtypeStruct((_B * _M, _NWORD), jnp.int32),
    )(xyzT, new_xyz, pk)


_NW = 32
_PER_W = _BT // _NW
_QW = (_B * _M) // _NW
_CH = 512

_PCB_NP = np.array([bin(x).count("1") for x in range(256)], np.int32)
_BSL_NP = np.zeros((2048,), np.int32)
for _byte in range(256):
    _r = 0
    for _s in range(8):
        if (_byte >> _s) & 1:
            _BSL_NP[_byte * 8 + _r] = _s
            _r += 1


@functools.lru_cache(maxsize=1)
def _make_select_gather():
    @functools.partial(
        pl.kernel,
        out_type=jax.ShapeDtypeStruct((_BT, _P), jnp.float32),
        mesh=plsc.VectorSubcoreMesh(core_axis_name="c", subcore_axis_name="s"),
        compiler_params=pltpu.CompilerParams(needs_layout_passes=False),
        scratch_types=[
            pltpu.VMEM((_QW * _NWORD,), jnp.int32),
            pltpu.VMEM((256,), jnp.int32),
            pltpu.VMEM((2048,), jnp.int32),
            pltpu.VMEM((_NWORD,), jnp.int32),
            pltpu.VMEM((_QW * _NS,), jnp.int32),
            pltpu.VMEM((_CH, _P), jnp.float32),
            pltpu.SemaphoreType.DMA,
        ],
    )
    def selgather(words_hbm, table_hbm, pcb_hbm, bsl_hbm, out_hbm,
                  words_v, pcb_v, bsl_v, p_v, idxbuf_v, rows_v, sem):
        wid = lax.axis_index("s") * 2 + lax.axis_index("c")
        qbase = wid * _QW
        pltpu.sync_copy(pcb_hbm, pcb_v)
        pltpu.sync_copy(bsl_hbm, bsl_v)
        pltpu.sync_copy(words_hbm.at[pl.ds(qbase * _NWORD, _QW * _NWORD)],
                        words_v)
        boff = (wid >> 3) << 12
        lanes = lax.iota(jnp.int32, 16)

        def one_query(qi, carry_unused):
            qo = qi << 8
            tot = jnp.int32(0)
            for v in range(_NWORD // 16):
                wv = words_v[pl.ds(qo + v * 16, 16)]
                cnt = (plsc.load_gather(pcb_v, [wv & 255])
                       + plsc.load_gather(pcb_v, [wv >> 8]))
                incl = plsc.cumsum(cnt)
                p_v[pl.ds(v * 16, 16)] = incl - cnt + tot
                tot = tot + jnp.max(incl)
            idxs = []
            kvs = []
            for half in range(2):
                kv = lanes + half * 16
                lo_b = jnp.zeros((16,), jnp.int32)
                hi_b = jnp.broadcast_to(jnp.int32(_NWORD), (16,))
                for _ in range(8):
                    mid = (lo_b + hi_b) >> 1
                    cond = plsc.load_gather(p_v, [mid]) <= kv
                    lo_b = jnp.where(cond, mid, lo_b)
                    hi_b = jnp.where(cond, hi_b, mid)
                ww = lo_b
                r = kv - plsc.load_gather(p_v, [ww])
                wv = plsc.load_gather(words_v, [ww + qo])
                lob = wv & 255
                pclo = plsc.load_gather(pcb_v, [lob])
                usehi = r >= pclo
                byte = jnp.where(usehi, wv >> 8, lob)
                r2 = jnp.minimum(jnp.where(usehi, r - pclo, r), 7)
                pos = (plsc.load_gather(bsl_v, [(byte << 3) + r2])
                       + jnp.where(usehi, 8, 0))
                idxs.append((ww << 4) + pos)
                kvs.append(kv)
            idx0 = jnp.sum(jnp.where(lanes == 0, idxs[0], 0))
            idx0 = jnp.where(tot > 0, idx0, 0)
            idx0v = jnp.broadcast_to(idx0, (16,))
            totv = jnp.broadcast_to(tot, (16,))
            for half in range(2):
                sel = jnp.where(kvs[half] < totv, idxs[half], idx0v) + boff
                idxbuf_v[pl.ds((qi << 5) + half * 16, 16)] = sel
            return carry_unused

        lax.fori_loop(0, _QW, one_query, jnp.int32(0))

        for ch in range(_PER_W // _CH):
            src = table_hbm.at[idxbuf_v.at[pl.ds(ch * _CH, _CH)]]
            pltpu.async_copy(src, rows_v, sem).wait()
            pltpu.sync_copy(rows_v,
                            out_hbm.at[pl.ds(wid * _PER_W + ch * _CH, _CH)])

    return selgather


def _select_gather(words_flat, table, pcb, bsl):
    return _make_select_gather()(words_flat, table, pcb, bsl)


def _k1_body(xg_ref, nq_ref, w1_ref, w1x_ref, b1_ref, y1_ref, s_ref, ss_ref):
    i = pl.program_id(0)
    xg = xg_ref[...]
    q = nq_ref[...]
    corr = jnp.dot(q, w1x_ref[...], preferred_element_type=jnp.float32)
    ri = lax.broadcasted_iota(jnp.int32, (_PB, _QB), 0)
    cj = lax.broadcasted_iota(jnp.int32, (_PB, _QB), 1)
    rep = (ri // _NS == cj).astype(jnp.float32)
    corr_rep = jnp.dot(rep, corr, preferred_element_type=jnp.float32)
    y1 = (jnp.dot(xg, w1_ref[...], preferred_element_type=jnp.float32)
          + b1_ref[...] - corr_rep)
    y1_ref[...] = y1
    ps = jnp.sum(y1, axis=0, keepdims=True)
    pss = jnp.sum(y1 * y1, axis=0, keepdims=True)

    @pl.when(i == 0)
    def _init():
        s_ref[...] = ps
        ss_ref[...] = pss

    @pl.when(i > 0)
    def _acc():
        s_ref[...] += ps
        ss_ref[...] += pss


def _k2_body(y1_ref, s1_ref, ss1_ref, g1_ref, bt1_ref, w2_ref, b2_ref,
             y2_ref, s_ref, ss_ref):
    i = pl.program_id(0)
    inv_n = jnp.float32(1.0 / _BT)
    m1 = s1_ref[...] * inv_n
    var1 = ss1_ref[...] * inv_n - m1 * m1
    sc = g1_ref[...] / jnp.sqrt(var1 + 1e-5)
    z = jnp.maximum((y1_ref[...] - m1) * sc + bt1_ref[...], 0.0)
    y2 = (jnp.dot(z, w2_ref[...], preferred_element_type=jnp.float32)
          + b2_ref[...])
    y2_ref[...] = y2
    ps = jnp.sum(y2, axis=0, keepdims=True)
    pss = jnp.sum(y2 * y2, axis=0, keepdims=True)

    @pl.when(i == 0)
    def _init():
        s_ref[...] = ps
        ss_ref[...] = pss

    @pl.when(i > 0)
    def _acc():
        s_ref[...] += ps
        ss_ref[...] += pss


def _k3_body(y2_ref, s2_ref, ss2_ref, g2_ref, bt2_ref, out_ref):
    inv_n = jnp.float32(1.0 / _BT)
    m2 = s2_ref[...] * inv_n
    var2 = ss2_ref[...] * inv_n - m2 * m2
    sc = g2_ref[...] / jnp.sqrt(var2 + 1e-5)
    y = y2_ref[...]
    z = jnp.maximum((y - m2) * sc + bt2_ref[...], 0.0)
    red = z[:, 0, :]
    for k in range(1, _NS):
        red = jnp.maximum(red, z[:, k, :])
    out_ref[...] = red


def _whole(shape):
    return pl.BlockSpec(shape, lambda i: tuple(0 for _ in shape))


def kernel(xyz, new_xyz, features, W1, b1, g1, bt1, W2, b2, g2, bt2):
    xyzT = jnp.transpose(xyz, (0, 2, 1))
    ftT = jnp.transpose(features, (0, 2, 1))
    pad = jnp.zeros((_B, _N, _P - 3 - _C), jnp.float32)
    table = jnp.concatenate([xyz, ftT, pad], axis=-1).reshape(_B * _N, _P)

    j = np.arange(_N)
    pk = jnp.asarray(
        ((j[:, None] // 16 == np.arange(_NWORD)[None, :])
         * (1 << (j % 16))[:, None]).astype(np.float32), jnp.bfloat16)
    words = _pack_words(xyzT, new_xyz, pk)
    xg = _select_gather(words.reshape(_B * _M * _NWORD),
                        table, jnp.asarray(_PCB_NP),
                        jnp.asarray(_BSL_NP))

    nq_flat = new_xyz.reshape(_B * _M, 3)
    w1p = jnp.pad(W1, ((0, 0), (0, _P - 3 - _C))).T
    w1x = W1[:, :3].T

    nblk = _BT // _PB
    y1, s1, ss1 = pl.pallas_call(
        _k1_body,
        grid=(nblk,),
        in_specs=[
            pl.BlockSpec((_PB, _P), lambda i: (i, 0)),
            pl.BlockSpec((_QB, 3), lambda i: (i, 0)),
            _whole((_P, _H1)),
            _whole((3, _H1)),
            _whole((1, _H1)),
        ],
        out_specs=[
            pl.BlockSpec((_PB, _H1), lambda i: (i, 0)),
            _whole((1, _H1)),
            _whole((1, _H1)),
        ],
        out_shape=[
            jax.ShapeDtypeStruct((_BT, _H1), jnp.float32),
            jax.ShapeDtypeStruct((1, _H1), jnp.float32),
            jax.ShapeDtypeStruct((1, _H1), jnp.float32),
        ],
    )(xg, nq_flat, w1p, w1x, b1[None])

    y2, s2, ss2 = pl.pallas_call(
        _k2_body,
        grid=(nblk,),
        in_specs=[
            pl.BlockSpec((_PB, _H1), lambda i: (i, 0)),
            _whole((1, _H1)),
            _whole((1, _H1)),
            _whole((1, _H1)),
            _whole((1, _H1)),
            _whole((_H1, _H2)),
            _whole((1, _H2)),
        ],
        out_specs=[
            pl.BlockSpec((_PB, _H2), lambda i: (i, 0)),
            _whole((1, _H2)),
            _whole((1, _H2)),
        ],
        out_shape=[
            jax.ShapeDtypeStruct((_BT, _H2), jnp.float32),
            jax.ShapeDtypeStruct((1, _H2), jnp.float32),
            jax.ShapeDtypeStruct((1, _H2), jnp.float32),
        ],
    )(y1, s1, ss1, g1[None], bt1[None], W2.T, b2[None])

    pooled = pl.pallas_call(
        _k3_body,
        grid=(_B * _M // _QB,),
        in_specs=[
            pl.BlockSpec((_QB, _NS, _H2), lambda i: (i, 0, 0)),
            _whole((1, _H2)),
            _whole((1, _H2)),
            _whole((1, _H2)),
            _whole((1, _H2)),
        ],
        out_specs=pl.BlockSpec((_QB, _H2), lambda i: (i, 0)),
        out_shape=jax.ShapeDtypeStruct((_B * _M, _H2), jnp.float32),
    )(y2.reshape(_B * _M, _NS, _H2), s2, ss2, g2[None], bt2[None])

    out = pooled.reshape(_B, _M, _H2).transpose(0, 2, 1)
    return (new_xyz, out)


# speedup vs baseline: 17.4647x; 1.1097x over previous
import functools

import jax
import jax.numpy as jnp
import numpy as np
from jax import lax
from jax.experimental import pallas as pl
from jax.experimental.pallas import tpu as pltpu
from jax.experimental.pallas import tpu_sc as plsc

_B, _N, _M, _C, _NS = 4, 4096, 1024, 64, 32
_R2 = 0.12 ** 2
_H1, _H2 = 64, 128
_P = 128
_BT = _B * _M * _NS
_MB = 256
_NCH = _N // 128
_QB = 64
_PB = _QB * _NS


_NWORD = _N // 16


def _pack_body(xyzT_ref, nq_ref, pk_ref, out_ref):
    t = xyzT_ref[...]
    q = nq_ref[...][0]
    p3 = t[0]
    pn = (p3[0:1, :] * p3[0:1, :] + p3[1:2, :] * p3[1:2, :]
          + p3[2:3, :] * p3[2:3, :])
    qn = (q[:, 0:1] * q[:, 0:1] + q[:, 1:2] * q[:, 1:2]
          + q[:, 2:3] * q[:, 2:3])
    lhs = jnp.concatenate([q, qn, jnp.ones((_MB, 1), jnp.float32)], axis=1)
    rhs = jnp.concatenate([-2.0 * p3, jnp.ones((1, _N), jnp.float32), pn],
                          axis=0)
    d2 = jnp.dot(lhs, rhs, precision=lax.Precision.HIGHEST,
                 preferred_element_type=jnp.float32)
    m = (d2 <= _R2).astype(jnp.bfloat16)
    words = jnp.dot(m, pk_ref[...], preferred_element_type=jnp.float32)
    out_ref[...] = words.astype(jnp.int32)


def _pack_words(xyzT, new_xyz, pk):
    return pl.pallas_call(
        _pack_body,
        grid=(_B, _M // _MB),
        in_specs=[
            pl.BlockSpec((1, 3, _N), lambda b, i: (b, 0, 0)),
            pl.BlockSpec((1, _MB, 3), lambda b, i: (b, i, 0)),
            pl.BlockSpec((_N, _NWORD), lambda b, i: (0, 0)),
        ],
        out_specs=pl.BlockSpec((_MB, _NWORD),
                               lambda b, i: (b * (_M // _MB) + i, 0)),
        out_shape=jax.ShapeDtypeStruct((_B * _M, _NWORD), jnp.int32),
    )(xyzT, new_xyz, pk)


_NW = 32
_PER_W = _BT // _NW
_QW = (_B * _M) // _NW
_CH = 256
_NG = _PER_W // _CH
_QG = _QW // _NG

_PCB_NP = np.array([bin(x).count("1") for x in range(256)], np.int32)
_BSL_NP = np.zeros((2048,), np.int32)
for _byte in range(256):
    _r = 0
    for _s in range(8):
        if (_byte >> _s) & 1:
            _BSL_NP[_byte * 8 + _r] = _s
            _r += 1


@functools.lru_cache(maxsize=1)
def _make_select_gather():
    @functools.partial(
        pl.kernel,
        out_type=jax.ShapeDtypeStruct((_BT, _P), jnp.float32),
        mesh=plsc.VectorSubcoreMesh(core_axis_name="c", subcore_axis_name="s"),
        compiler_params=pltpu.CompilerParams(needs_layout_passes=False),
        scratch_types=[
            pltpu.VMEM((_QW * _NWORD,), jnp.int32),
            pltpu.VMEM((256,), jnp.int32),
            pltpu.VMEM((2048,), jnp.int32),
            pltpu.VMEM((_NWORD,), jnp.int32),
            pltpu.VMEM((_QW * _NS,), jnp.int32),
            pltpu.VMEM((2, _CH, _P), jnp.float32),
            pltpu.SemaphoreType.DMA,
            pltpu.SemaphoreType.DMA,
        ],
    )
    def selgather(words_hbm, table_hbm, pcb_hbm, bsl_hbm, out_hbm,
                  words_v, pcb_v, bsl_v, p_v, idxbuf_v, rows_v, sem_g, sem_s):
        wid = lax.axis_index("s") * 2 + lax.axis_index("c")
        qbase = wid * _QW
        pltpu.sync_copy(pcb_hbm, pcb_v)
        pltpu.sync_copy(bsl_hbm, bsl_v)
        pltpu.sync_copy(words_hbm.at[pl.ds(qbase * _NWORD, _QW * _NWORD)],
                        words_v)
        boff = (wid >> 3) << 12
        lanes = lax.iota(jnp.int32, 16)

        def one_query(qi, carry_unused):
            qo = qi << 8
            tot = jnp.int32(0)
            for v in range(_NWORD // 16):
                wv = words_v[pl.ds(qo + v * 16, 16)]
                cnt = (plsc.load_gather(pcb_v, [wv & 255])
                       + plsc.load_gather(pcb_v, [wv >> 8]))
                incl = plsc.cumsum(cnt)
                p_v[pl.ds(v * 16, 16)] = incl - cnt + tot
                tot = tot + jnp.max(incl)
            idxs = []
            kvs = []
            for half in range(2):
                kv = lanes + half * 16
                lo_b = jnp.zeros((16,), jnp.int32)
                hi_b = jnp.broadcast_to(jnp.int32(_NWORD), (16,))
                for _ in range(8):
                    mid = (lo_b + hi_b) >> 1
                    cond = plsc.load_gather(p_v, [mid]) <= kv
                    lo_b = jnp.where(cond, mid, lo_b)
                    hi_b = jnp.where(cond, hi_b, mid)
                ww = lo_b
                r = kv - plsc.load_gather(p_v, [ww])
                wv = plsc.load_gather(words_v, [ww + qo])
                lob = wv & 255
                pclo = plsc.load_gather(pcb_v, [lob])
                usehi = r >= pclo
                byte = jnp.where(usehi, wv >> 8, lob)
                r2 = jnp.minimum(jnp.where(usehi, r - pclo, r), 7)
                pos = (plsc.load_gather(bsl_v, [(byte << 3) + r2])
                       + jnp.where(usehi, 8, 0))
                idxs.append((ww << 4) + pos)
                kvs.append(kv)
            idx0 = jnp.sum(jnp.where(lanes == 0, idxs[0], 0))
            idx0 = jnp.where(tot > 0, idx0, 0)
            idx0v = jnp.broadcast_to(idx0, (16,))
            totv = jnp.broadcast_to(tot, (16,))
            for half in range(2):
                sel = jnp.where(kvs[half] < totv, idxs[half], idx0v) + boff
                idxbuf_v[pl.ds((qi << 5) + half * 16, 16)] = sel
            return carry_unused

        obase = wid * _PER_W
        bufs = [rows_v.at[0], rows_v.at[1]]
        gh = [None] * _NG
        sh = [None] * _NG
        for g in range(_NG):
            lax.fori_loop(g * _QG, (g + 1) * _QG, one_query, jnp.int32(0))
            if g >= 1:
                gh[g - 1].wait()
                sh[g - 1] = pltpu.async_copy(
                    bufs[(g - 1) & 1],
                    out_hbm.at[pl.ds(obase + (g - 1) * _CH, _CH)], sem_s)
            if g >= 2:
                sh[g - 2].wait()
            gh[g] = pltpu.async_copy(
                table_hbm.at[idxbuf_v.at[pl.ds(g * _CH, _CH)]],
                bufs[g & 1], sem_g)
        gh[_NG - 1].wait()
        sh[_NG - 1] = pltpu.async_copy(
            bufs[(_NG - 1) & 1],
            out_hbm.at[pl.ds(obase + (_NG - 1) * _CH, _CH)], sem_s)
        sh[_NG - 2].wait()
        sh[_NG - 1].wait()

    return selgather


def _select_gather(words_flat, table, pcb, bsl):
    return _make_select_gather()(words_flat, table, pcb, bsl)


def _k1_body(xg_ref, nq_ref, w1_ref, w1x_ref, b1_ref, y1_ref, s_ref, ss_ref):
    i = pl.program_id(0)
    xg = xg_ref[...]
    q = nq_ref[...]
    corr = jnp.dot(q, w1x_ref[...], preferred_element_type=jnp.float32)
    ri = lax.broadcasted_iota(jnp.int32, (_PB, _QB), 0)
    cj = lax.broadcasted_iota(jnp.int32, (_PB, _QB), 1)
    rep = (ri // _NS == cj).astype(jnp.float32)
    corr_rep = jnp.dot(rep, corr, preferred_element_type=jnp.float32)
    y1 = (jnp.dot(xg, w1_ref[...], preferred_element_type=jnp.float32)
          + b1_ref[...] - corr_rep)
    y1_ref[...] = y1
    ps = jnp.sum(y1, axis=0, keepdims=True)
    pss = jnp.sum(y1 * y1, axis=0, keepdims=True)

    @pl.when(i == 0)
    def _init():
        s_ref[...] = ps
        ss_ref[...] = pss

    @pl.when(i > 0)
    def _acc():
        s_ref[...] += ps
        ss_ref[...] += pss


def _k2_body(y1_ref, s1_ref, ss1_ref, g1_ref, bt1_ref, w2_ref, b2_ref,
             y2_ref, s_ref, ss_ref):
    i = pl.program_id(0)
    inv_n = jnp.float32(1.0 / _BT)
    m1 = s1_ref[...] * inv_n
    var1 = ss1_ref[...] * inv_n - m1 * m1
    sc = g1_ref[...] / jnp.sqrt(var1 + 1e-5)
    z = jnp.maximum((y1_ref[...] - m1) * sc + bt1_ref[...], 0.0)
    y2 = (jnp.dot(z, w2_ref[...], preferred_element_type=jnp.float32)
          + b2_ref[...])
    y2_ref[...] = y2
    ps = jnp.sum(y2, axis=0, keepdims=True)
    pss = jnp.sum(y2 * y2, axis=0, keepdims=True)

    @pl.when(i == 0)
    def _init():
        s_ref[...] = ps
        ss_ref[...] = pss

    @pl.when(i > 0)
    def _acc():
        s_ref[...] += ps
        ss_ref[...] += pss


def _k3_body(y2_ref, s2_ref, ss2_ref, g2_ref, bt2_ref, out_ref):
    inv_n = jnp.float32(1.0 / _BT)
    m2 = s2_ref[...] * inv_n
    var2 = ss2_ref[...] * inv_n - m2 * m2
    sc = g2_ref[...] / jnp.sqrt(var2 + 1e-5)
    y = y2_ref[...]
    z = jnp.maximum((y - m2) * sc + bt2_ref[...], 0.0)
    red = z[:, 0, :]
    for k in range(1, _NS):
        red = jnp.maximum(red, z[:, k, :])
    out_ref[...] = red


def _whole(shape):
    return pl.BlockSpec(shape, lambda i: tuple(0 for _ in shape))


def kernel(xyz, new_xyz, features, W1, b1, g1, bt1, W2, b2, g2, bt2):
    xyzT = jnp.transpose(xyz, (0, 2, 1))
    ftT = jnp.transpose(features, (0, 2, 1))
    pad = jnp.zeros((_B, _N, _P - 3 - _C), jnp.float32)
    table = jnp.concatenate([xyz, ftT, pad], axis=-1).reshape(_B * _N, _P)

    j = np.arange(_N)
    pk = jnp.asarray(
        ((j[:, None] // 16 == np.arange(_NWORD)[None, :])
         * (1 << (j % 16))[:, None]).astype(np.float32), jnp.bfloat16)
    words = _pack_words(xyzT, new_xyz, pk)
    xg = _select_gather(words.reshape(_B * _M * _NWORD),
                        table, jnp.asarray(_PCB_NP),
                        jnp.asarray(_BSL_NP))

    nq_flat = new_xyz.reshape(_B * _M, 3)
    w1p = jnp.pad(W1, ((0, 0), (0, _P - 3 - _C))).T
    w1x = W1[:, :3].T

    nblk = _BT // _PB
    y1, s1, ss1 = pl.pallas_call(
        _k1_body,
        grid=(nblk,),
        in_specs=[
            pl.BlockSpec((_PB, _P), lambda i: (i, 0)),
            pl.BlockSpec((_QB, 3), lambda i: (i, 0)),
            _whole((_P, _H1)),
            _whole((3, _H1)),
            _whole((1, _H1)),
        ],
        out_specs=[
            pl.BlockSpec((_PB, _H1), lambda i: (i, 0)),
            _whole((1, _H1)),
            _whole((1, _H1)),
        ],
        out_shape=[
            jax.ShapeDtypeStruct((_BT, _H1), jnp.float32),
            jax.ShapeDtypeStruct((1, _H1), jnp.float32),
            jax.ShapeDtypeStruct((1, _H1), jnp.float32),
        ],
    )(xg, nq_flat, w1p, w1x, b1[None])

    y2, s2, ss2 = pl.pallas_call(
        _k2_body,
        grid=(nblk,),
        in_specs=[
            pl.BlockSpec((_PB, _H1), lambda i: (i, 0)),
            _whole((1, _H1)),
            _whole((1, _H1)),
            _whole((1, _H1)),
            _whole((1, _H1)),
            _whole((_H1, _H2)),
            _whole((1, _H2)),
        ],
        out_specs=[
            pl.BlockSpec((_PB, _H2), lambda i: (i, 0)),
            _whole((1, _H2)),
            _whole((1, _H2)),
        ],
        out_shape=[
            jax.ShapeDtypeStruct((_BT, _H2), jnp.float32),
            jax.ShapeDtypeStruct((1, _H2), jnp.float32),
            jax.ShapeDtypeStruct((1, _H2), jnp.float32),
        ],
    )(y1, s1, ss1, g1[None], bt1[None], W2.T, b2[None])

    pooled = pl.pallas_call(
        _k3_body,
        grid=(_B * _M // _QB,),
        in_specs=[
            pl.BlockSpec((_QB, _NS, _H2), lambda i: (i, 0, 0)),
            _whole((1, _H2)),
            _whole((1, _H2)),
            _whole((1, _H2)),
            _whole((1, _H2)),
        ],
        out_specs=pl.BlockSpec((_QB, _H2), lambda i: (i, 0)),
        out_shape=jax.ShapeDtypeStruct((_B * _M, _H2), jnp.float32),
    )(y2.reshape(_B * _M, _NS, _H2), s2, ss2, g2[None], bt2[None])

    out = pooled.reshape(_B, _M, _H2).transpose(0, 2, 1)
    return (new_xyz, out)


# speedup vs baseline: 19.5975x; 1.1221x over previous
import functools

import jax
import jax.numpy as jnp
import numpy as np
from jax import lax
from jax.experimental import pallas as pl
from jax.experimental.pallas import tpu as pltpu
from jax.experimental.pallas import tpu_sc as plsc

_B, _N, _M, _C, _NS = 4, 4096, 1024, 64, 32
_R2 = 0.12 ** 2
_H1, _H2 = 64, 128
_P = 128
_BT = _B * _M * _NS
_MB = 512
_NCH = _N // 128
_QB = 64
_PB = _QB * _NS


_NWORD = _N // 16


def _pack_body(xyzT_ref, nq_ref, pk_ref, out_ref):
    t = xyzT_ref[...]
    q = nq_ref[...][0]
    dx = q[:, 0:1] - t[0, 0:1, :]
    dy = q[:, 1:2] - t[0, 1:2, :]
    dz = q[:, 2:3] - t[0, 2:3, :]
    d2 = dx * dx + dy * dy + dz * dz
    m = (d2 <= _R2).astype(jnp.bfloat16)
    words = jnp.dot(m, pk_ref[...], preferred_element_type=jnp.float32)
    out_ref[...] = words.astype(jnp.int32)


def _pack_words(xyzT, new_xyz, pk):
    return pl.pallas_call(
        _pack_body,
        grid=(_B, _M // _MB),
        in_specs=[
            pl.BlockSpec((1, 3, _N), lambda b, i: (b, 0, 0)),
            pl.BlockSpec((1, _MB, 3), lambda b, i: (b, i, 0)),
            pl.BlockSpec((_N, _NWORD), lambda b, i: (0, 0)),
        ],
        out_specs=pl.BlockSpec((_MB, _NWORD),
                               lambda b, i: (b * (_M // _MB) + i, 0)),
        out_shape=jax.ShapeDtypeStruct((_B * _M, _NWORD), jnp.int32),
    )(xyzT, new_xyz, pk)


_NW = 32
_PER_W = _BT // _NW
_QW = (_B * _M) // _NW
_CH = 256
_NG = _PER_W // _CH
_QG = _QW // _NG

_PCB_NP = np.array([bin(x).count("1") for x in range(256)], np.int32)
_BSL_NP = np.zeros((2048,), np.int32)
for _byte in range(256):
    _r = 0
    for _s in range(8):
        if (_byte >> _s) & 1:
            _BSL_NP[_byte * 8 + _r] = _s
            _r += 1


@functools.lru_cache(maxsize=1)
def _make_select_gather():
    @functools.partial(
        pl.kernel,
        out_type=jax.ShapeDtypeStruct((_BT, _P), jnp.float32),
        mesh=plsc.VectorSubcoreMesh(core_axis_name="c", subcore_axis_name="s"),
        compiler_params=pltpu.CompilerParams(needs_layout_passes=False),
        scratch_types=[
            pltpu.VMEM((_QW * _NWORD,), jnp.int32),
            pltpu.VMEM((256,), jnp.int32),
            pltpu.VMEM((2048,), jnp.int32),
            pltpu.VMEM((_NWORD,), jnp.int32),
            pltpu.VMEM((_QW * _NS,), jnp.int32),
            pltpu.VMEM((2, _CH, _P), jnp.float32),
            pltpu.SemaphoreType.DMA,
            pltpu.SemaphoreType.DMA,
        ],
    )
    def selgather(words_hbm, table_hbm, pcb_hbm, bsl_hbm, out_hbm,
                  words_v, pcb_v, bsl_v, p_v, idxbuf_v, rows_v, sem_g, sem_s):
        wid = lax.axis_index("s") * 2 + lax.axis_index("c")
        qbase = wid * _QW
        pltpu.sync_copy(pcb_hbm, pcb_v)
        pltpu.sync_copy(bsl_hbm, bsl_v)
        pltpu.sync_copy(words_hbm.at[pl.ds(qbase * _NWORD, _QW * _NWORD)],
                        words_v)
        boff = (wid >> 3) << 12
        lanes = lax.iota(jnp.int32, 16)

        def one_query(qi, carry_unused):
            qo = qi << 8
            tot = jnp.int32(0)
            for v in range(_NWORD // 16):
                wv = words_v[pl.ds(qo + v * 16, 16)]
                cnt = (plsc.load_gather(pcb_v, [wv & 255])
                       + plsc.load_gather(pcb_v, [wv >> 8]))
                incl = plsc.cumsum(cnt)
                p_v[pl.ds(v * 16, 16)] = incl - cnt + tot
                tot = tot + jnp.max(incl)
            idxs = []
            kvs = []
            for half in range(2):
                kv = lanes + half * 16
                lo_b = jnp.zeros((16,), jnp.int32)
                hi_b = jnp.broadcast_to(jnp.int32(_NWORD), (16,))
                for _ in range(8):
                    mid = (lo_b + hi_b) >> 1
                    cond = plsc.load_gather(p_v, [mid]) <= kv
                    lo_b = jnp.where(cond, mid, lo_b)
                    hi_b = jnp.where(cond, hi_b, mid)
                ww = lo_b
                r = kv - plsc.load_gather(p_v, [ww])
                wv = plsc.load_gather(words_v, [ww + qo])
                lob = wv & 255
                pclo = plsc.load_gather(pcb_v, [lob])
                usehi = r >= pclo
                byte = jnp.where(usehi, wv >> 8, lob)
                r2 = jnp.minimum(jnp.where(usehi, r - pclo, r), 7)
                pos = (plsc.load_gather(bsl_v, [(byte << 3) + r2])
                       + jnp.where(usehi, 8, 0))
                idxs.append((ww << 4) + pos)
                kvs.append(kv)
            idx0 = jnp.sum(jnp.where(lanes == 0, idxs[0], 0))
            idx0 = jnp.where(tot > 0, idx0, 0)
            idx0v = jnp.broadcast_to(idx0, (16,))
            totv = jnp.broadcast_to(tot, (16,))
            for half in range(2):
                sel = jnp.where(kvs[half] < totv, idxs[half], idx0v) + boff
                idxbuf_v[pl.ds((qi << 5) + half * 16, 16)] = sel
            return carry_unused

        obase = wid * _PER_W
        bufs = [rows_v.at[0], rows_v.at[1]]
        gh = [None] * _NG
        sh = [None] * _NG
        for g in range(_NG):
            lax.fori_loop(g * _QG, (g + 1) * _QG, one_query, jnp.int32(0))
            if g >= 1:
                gh[g - 1].wait()
                sh[g - 1] = pltpu.async_copy(
                    bufs[(g - 1) & 1],
                    out_hbm.at[pl.ds(obase + (g - 1) * _CH, _CH)], sem_s)
            if g >= 2:
                sh[g - 2].wait()
            gh[g] = pltpu.async_copy(
                table_hbm.at[idxbuf_v.at[pl.ds(g * _CH, _CH)]],
                bufs[g & 1], sem_g)
        gh[_NG - 1].wait()
        sh[_NG - 1] = pltpu.async_copy(
            bufs[(_NG - 1) & 1],
            out_hbm.at[pl.ds(obase + (_NG - 1) * _CH, _CH)], sem_s)
        sh[_NG - 2].wait()
        sh[_NG - 1].wait()

    return selgather


def _select_gather(words_flat, table, pcb, bsl):
    return _make_select_gather()(words_flat, table, pcb, bsl)


def _k1_body(xg_ref, nq_ref, w1_ref, w1x_ref, b1_ref, y1_ref, s_ref, ss_ref):
    i = pl.program_id(0)
    xg = xg_ref[...]
    q = nq_ref[...]
    corr = jnp.dot(q, w1x_ref[...], preferred_element_type=jnp.float32)
    ri = lax.broadcasted_iota(jnp.int32, (_PB, _QB), 0)
    cj = lax.broadcasted_iota(jnp.int32, (_PB, _QB), 1)
    rep = (ri // _NS == cj).astype(jnp.float32)
    corr_rep = jnp.dot(rep, corr, preferred_element_type=jnp.float32)
    y1 = (jnp.dot(xg, w1_ref[...], preferred_element_type=jnp.float32)
          + b1_ref[...] - corr_rep)
    y1_ref[...] = y1
    ps = jnp.sum(y1, axis=0, keepdims=True)
    pss = jnp.sum(y1 * y1, axis=0, keepdims=True)

    @pl.when(i == 0)
    def _init():
        s_ref[...] = ps
        ss_ref[...] = pss

    @pl.when(i > 0)
    def _acc():
        s_ref[...] += ps
        ss_ref[...] += pss


def _k2_body(y1_ref, s1_ref, ss1_ref, g1_ref, bt1_ref, w2_ref, b2_ref,
             y2_ref, s_ref, ss_ref):
    i = pl.program_id(0)
    inv_n = jnp.float32(1.0 / _BT)
    m1 = s1_ref[...] * inv_n
    var1 = ss1_ref[...] * inv_n - m1 * m1
    sc = g1_ref[...] / jnp.sqrt(var1 + 1e-5)
    z = jnp.maximum((y1_ref[...] - m1) * sc + bt1_ref[...], 0.0)
    y2 = (jnp.dot(z, w2_ref[...], preferred_element_type=jnp.float32)
          + b2_ref[...])
    y2_ref[...] = y2
    ps = jnp.sum(y2, axis=0, keepdims=True)
    pss = jnp.sum(y2 * y2, axis=0, keepdims=True)

    @pl.when(i == 0)
    def _init():
        s_ref[...] = ps
        ss_ref[...] = pss

    @pl.when(i > 0)
    def _acc():
        s_ref[...] += ps
        ss_ref[...] += pss


def _k3_body(y2_ref, s2_ref, ss2_ref, g2_ref, bt2_ref, out_ref):
    inv_n = jnp.float32(1.0 / _BT)
    m2 = s2_ref[...] * inv_n
    var2 = ss2_ref[...] * inv_n - m2 * m2
    sc = g2_ref[...] / jnp.sqrt(var2 + 1e-5)
    y = y2_ref[...]
    z = jnp.maximum((y - m2) * sc + bt2_ref[...], 0.0)
    red = z[:, 0, :]
    for k in range(1, _NS):
        red = jnp.maximum(red, z[:, k, :])
    out_ref[...] = red


def _whole(shape):
    return pl.BlockSpec(shape, lambda i: tuple(0 for _ in shape))


def kernel(xyz, new_xyz, features, W1, b1, g1, bt1, W2, b2, g2, bt2):
    xyzT = jnp.transpose(xyz, (0, 2, 1))
    ftT = jnp.transpose(features, (0, 2, 1))
    pad = jnp.zeros((_B, _N, _P - 3 - _C), jnp.float32)
    table = jnp.concatenate([xyz, ftT, pad], axis=-1).reshape(_B * _N, _P)

    j = np.arange(_N)
    pk = jnp.asarray(
        ((j[:, None] // 16 == np.arange(_NWORD)[None, :])
         * (1 << (j % 16))[:, None]).astype(np.float32), jnp.bfloat16)
    words = _pack_words(xyzT, new_xyz, pk)
    xg = _select_gather(words.reshape(_B * _M * _NWORD),
                        table, jnp.asarray(_PCB_NP),
                        jnp.asarray(_BSL_NP))

    nq_flat = new_xyz.reshape(_B * _M, 3)
    w1p = jnp.pad(W1, ((0, 0), (0, _P - 3 - _C))).T
    w1x = W1[:, :3].T

    nblk = _BT // _PB
    y1, s1, ss1 = pl.pallas_call(
        _k1_body,
        grid=(nblk,),
        in_specs=[
            pl.BlockSpec((_PB, _P), lambda i: (i, 0)),
            pl.BlockSpec((_QB, 3), lambda i: (i, 0)),
            _whole((_P, _H1)),
            _whole((3, _H1)),
            _whole((1, _H1)),
        ],
        out_specs=[
            pl.BlockSpec((_PB, _H1), lambda i: (i, 0)),
            _whole((1, _H1)),
            _whole((1, _H1)),
        ],
        out_shape=[
            jax.ShapeDtypeStruct((_BT, _H1), jnp.float32),
            jax.ShapeDtypeStruct((1, _H1), jnp.float32),
            jax.ShapeDtypeStruct((1, _H1), jnp.float32),
        ],
    )(xg, nq_flat, w1p, w1x, b1[None])

    y2, s2, ss2 = pl.pallas_call(
        _k2_body,
        grid=(nblk,),
        in_specs=[
            pl.BlockSpec((_PB, _H1), lambda i: (i, 0)),
            _whole((1, _H1)),
            _whole((1, _H1)),
            _whole((1, _H1)),
            _whole((1, _H1)),
            _whole((_H1, _H2)),
            _whole((1, _H2)),
        ],
        out_specs=[
            pl.BlockSpec((_PB, _H2), lambda i: (i, 0)),
            _whole((1, _H2)),
            _whole((1, _H2)),
        ],
        out_shape=[
            jax.ShapeDtypeStruct((_BT, _H2), jnp.float32),
            jax.ShapeDtypeStruct((1, _H2), jnp.float32),
            jax.ShapeDtypeStruct((1, _H2), jnp.float32),
        ],
    )(y1, s1, ss1, g1[None], bt1[None], W2.T, b2[None])

    pooled = pl.pallas_call(
        _k3_body,
        grid=(_B * _M // _QB,),
        in_specs=[
            pl.BlockSpec((_QB, _NS, _H2), lambda i: (i, 0, 0)),
            _whole((1, _H2)),
            _whole((1, _H2)),
            _whole((1, _H2)),
            _whole((1, _H2)),
        ],
        out_specs=pl.BlockSpec((_QB, _H2), lambda i: (i, 0)),
        out_shape=jax.ShapeDtypeStruct((_B * _M, _H2), jnp.float32),
    )(y2.reshape(_B * _M, _NS, _H2), s2, ss2, g2[None], bt2[None])

    out = pooled.reshape(_B, _M, _H2).transpose(0, 2, 1)
    return (new_xyz, out)


# speedup vs baseline: 22.1559x; 1.1305x over previous
import functools

import jax
import jax.numpy as jnp
import numpy as np
from jax import lax
from jax.experimental import pallas as pl
from jax.experimental.pallas import tpu as pltpu
from jax.experimental.pallas import tpu_sc as plsc

_B, _N, _M, _C, _NS = 4, 4096, 1024, 64, 32
_R2 = 0.12 ** 2
_H1, _H2 = 64, 128
_P = 128
_BT = _B * _M * _NS
_MB = 512
_NCH = _N // 128
_QB = 128
_PB = _QB * _NS


_NWORD = _N // 16


def _pack_body(xyzT_ref, nq_ref, pk_ref, out_ref):
    t = xyzT_ref[...]
    q = nq_ref[...][0]
    dx = q[:, 0:1] - t[0, 0:1, :]
    dy = q[:, 1:2] - t[0, 1:2, :]
    dz = q[:, 2:3] - t[0, 2:3, :]
    d2 = dx * dx + dy * dy + dz * dz
    m = (d2 <= _R2).astype(jnp.bfloat16)
    words = jnp.dot(m, pk_ref[...], preferred_element_type=jnp.float32)
    out_ref[...] = words.astype(jnp.int32)


_TS = _N // (_M // _MB)


def _pack_body2(xyzT_ref, nq_ref, pk_ref, xyzs_ref, ft_ref, out_ref, tab_ref):
    _pack_body(xyzT_ref, nq_ref, pk_ref, out_ref)
    pslice = jnp.transpose(xyzs_ref[...][0])
    fslice = jnp.transpose(ft_ref[...][0])
    zpad = jnp.zeros((_TS, _P - 3 - _C), jnp.float32)
    tab_ref[...] = jnp.concatenate([pslice, fslice, zpad], axis=1)


def _pack_words(xyzT, new_xyz, pk, features):
    return pl.pallas_call(
        _pack_body2,
        grid=(_B, _M // _MB),
        in_specs=[
            pl.BlockSpec((1, 3, _N), lambda b, i: (b, 0, 0)),
            pl.BlockSpec((1, _MB, 3), lambda b, i: (b, i, 0)),
            pl.BlockSpec((_N, _NWORD), lambda b, i: (0, 0)),
            pl.BlockSpec((1, 3, _TS), lambda b, i: (b, 0, i)),
            pl.BlockSpec((1, _C, _TS), lambda b, i: (b, 0, i)),
        ],
        out_specs=[
            pl.BlockSpec((_MB, _NWORD),
                         lambda b, i: (b * (_M // _MB) + i, 0)),
            pl.BlockSpec((_TS, _P),
                         lambda b, i: (b * (_N // _TS) + i, 0)),
        ],
        out_shape=[
            jax.ShapeDtypeStruct((_B * _M, _NWORD), jnp.int32),
            jax.ShapeDtypeStruct((_B * _N, _P), jnp.float32),
        ],
    )(xyzT, new_xyz, pk, xyzT, features)


_NW = 32
_PER_W = _BT // _NW
_QW = (_B * _M) // _NW
_CH = 256
_NG = _PER_W // _CH
_QG = _QW // _NG

_PCB_NP = np.array([bin(x).count("1") for x in range(256)], np.int32)
_BSL_NP = np.zeros((2048,), np.int32)
for _byte in range(256):
    _r = 0
    for _s in range(8):
        if (_byte >> _s) & 1:
            _BSL_NP[_byte * 8 + _r] = _s
            _r += 1


@functools.lru_cache(maxsize=1)
def _make_select_gather():
    @functools.partial(
        pl.kernel,
        out_type=jax.ShapeDtypeStruct((_BT, _P), jnp.float32),
        mesh=plsc.VectorSubcoreMesh(core_axis_name="c", subcore_axis_name="s"),
        compiler_params=pltpu.CompilerParams(needs_layout_passes=False),
        scratch_types=[
            pltpu.VMEM((_QW * _NWORD,), jnp.int32),
            pltpu.VMEM((256,), jnp.int32),
            pltpu.VMEM((2048,), jnp.int32),
            pltpu.VMEM((_NWORD,), jnp.int32),
            pltpu.VMEM((_QW * _NS,), jnp.int32),
            pltpu.VMEM((2, _CH, _P), jnp.float32),
            pltpu.SemaphoreType.DMA,
            pltpu.SemaphoreType.DMA,
        ],
    )
    def selgather(words_hbm, table_hbm, pcb_hbm, bsl_hbm, out_hbm,
                  words_v, pcb_v, bsl_v, p_v, idxbuf_v, rows_v, sem_g, sem_s):
        wid = lax.axis_index("s") * 2 + lax.axis_index("c")
        qbase = wid * _QW
        pltpu.sync_copy(pcb_hbm, pcb_v)
        pltpu.sync_copy(bsl_hbm, bsl_v)
        pltpu.sync_copy(words_hbm.at[pl.ds(qbase * _NWORD, _QW * _NWORD)],
                        words_v)
        boff = (wid >> 3) << 12
        lanes = lax.iota(jnp.int32, 16)

        def one_query(qi, carry_unused):
            qo = qi << 8
            tot = jnp.int32(0)
            for v in range(_NWORD // 16):
                wv = words_v[pl.ds(qo + v * 16, 16)]
                cnt = (plsc.load_gather(pcb_v, [wv & 255])
                       + plsc.load_gather(pcb_v, [wv >> 8]))
                incl = plsc.cumsum(cnt)
                p_v[pl.ds(v * 16, 16)] = incl - cnt + tot
                tot = tot + jnp.max(incl)
            idxs = []
            kvs = []
            for half in range(2):
                kv = lanes + half * 16
                lo_b = jnp.zeros((16,), jnp.int32)
                hi_b = jnp.broadcast_to(jnp.int32(_NWORD), (16,))
                for _ in range(8):
                    mid = (lo_b + hi_b) >> 1
                    cond = plsc.load_gather(p_v, [mid]) <= kv
                    lo_b = jnp.where(cond, mid, lo_b)
                    hi_b = jnp.where(cond, hi_b, mid)
                ww = lo_b
                r = kv - plsc.load_gather(p_v, [ww])
                wv = plsc.load_gather(words_v, [ww + qo])
                lob = wv & 255
                pclo = plsc.load_gather(pcb_v, [lob])
                usehi = r >= pclo
                byte = jnp.where(usehi, wv >> 8, lob)
                r2 = jnp.minimum(jnp.where(usehi, r - pclo, r), 7)
                pos = (plsc.load_gather(bsl_v, [(byte << 3) + r2])
                       + jnp.where(usehi, 8, 0))
                idxs.append((ww << 4) + pos)
                kvs.append(kv)
            idx0 = jnp.sum(jnp.where(lanes == 0, idxs[0], 0))
            idx0 = jnp.where(tot > 0, idx0, 0)
            idx0v = jnp.broadcast_to(idx0, (16,))
            totv = jnp.broadcast_to(tot, (16,))
            for half in range(2):
                sel = jnp.where(kvs[half] < totv, idxs[half], idx0v) + boff
                idxbuf_v[pl.ds((qi << 5) + half * 16, 16)] = sel
            return carry_unused

        obase = wid * _PER_W
        bufs = [rows_v.at[0], rows_v.at[1]]
        gh = [None] * _NG
        sh = [None] * _NG
        for g in range(_NG):
            lax.fori_loop(g * _QG, (g + 1) * _QG, one_query, jnp.int32(0))
            if g >= 1:
                gh[g - 1].wait()
                sh[g - 1] = pltpu.async_copy(
                    bufs[(g - 1) & 1],
                    out_hbm.at[pl.ds(obase + (g - 1) * _CH, _CH)], sem_s)
            if g >= 2:
                sh[g - 2].wait()
            gh[g] = pltpu.async_copy(
                table_hbm.at[idxbuf_v.at[pl.ds(g * _CH, _CH)]],
                bufs[g & 1], sem_g)
        gh[_NG - 1].wait()
        sh[_NG - 1] = pltpu.async_copy(
            bufs[(_NG - 1) & 1],
            out_hbm.at[pl.ds(obase + (_NG - 1) * _CH, _CH)], sem_s)
        sh[_NG - 2].wait()
        sh[_NG - 1].wait()

    return selgather


def _select_gather(words_flat, table, pcb, bsl):
    return _make_select_gather()(words_flat, table, pcb, bsl)


def _k1_body(xg_ref, nq_ref, w1_ref, w1x_ref, b1_ref, y1_ref, s_ref, ss_ref):
    i = pl.program_id(0)
    xg = xg_ref[...]
    q = nq_ref[...]
    corr = jnp.dot(q, w1x_ref[...], preferred_element_type=jnp.float32)
    ri = lax.broadcasted_iota(jnp.int32, (_PB, _QB), 0)
    cj = lax.broadcasted_iota(jnp.int32, (_PB, _QB), 1)
    rep = (ri // _NS == cj).astype(jnp.float32)
    corr_rep = jnp.dot(rep, corr, preferred_element_type=jnp.float32)
    y1 = (jnp.dot(xg, w1_ref[...], preferred_element_type=jnp.float32)
          + b1_ref[...] - corr_rep)
    y1_ref[...] = y1
    ps = jnp.sum(y1, axis=0, keepdims=True)
    pss = jnp.sum(y1 * y1, axis=0, keepdims=True)

    @pl.when(i == 0)
    def _init():
        s_ref[...] = ps
        ss_ref[...] = pss

    @pl.when(i > 0)
    def _acc():
        s_ref[...] += ps
        ss_ref[...] += pss


def _k2_body(y1_ref, s1_ref, ss1_ref, g1_ref, bt1_ref, w2_ref, b2_ref,
             y2_ref, s_ref, ss_ref):
    i = pl.program_id(0)
    inv_n = jnp.float32(1.0 / _BT)
    m1 = s1_ref[...] * inv_n
    var1 = ss1_ref[...] * inv_n - m1 * m1
    sc = g1_ref[...] / jnp.sqrt(var1 + 1e-5)
    z = jnp.maximum((y1_ref[...] - m1) * sc + bt1_ref[...], 0.0)
    y2 = (jnp.dot(z, w2_ref[...], preferred_element_type=jnp.float32)
          + b2_ref[...])
    y2_ref[...] = y2
    ps = jnp.sum(y2, axis=0, keepdims=True)
    pss = jnp.sum(y2 * y2, axis=0, keepdims=True)

    @pl.when(i == 0)
    def _init():
        s_ref[...] = ps
        ss_ref[...] = pss

    @pl.when(i > 0)
    def _acc():
        s_ref[...] += ps
        ss_ref[...] += pss


def _k3_body(y2_ref, s2_ref, ss2_ref, g2_ref, bt2_ref, out_ref):
    inv_n = jnp.float32(1.0 / _BT)
    m2 = s2_ref[...] * inv_n
    var2 = ss2_ref[...] * inv_n - m2 * m2
    sc = g2_ref[...] / jnp.sqrt(var2 + 1e-5)
    y = y2_ref[...]
    z = jnp.maximum((y - m2) * sc + bt2_ref[...], 0.0)
    red = z[:, 0, :]
    for k in range(1, _NS):
        red = jnp.maximum(red, z[:, k, :])
    out_ref[...] = jnp.transpose(red)[None]


def _whole(shape):
    return pl.BlockSpec(shape, lambda i: tuple(0 for _ in shape))


def kernel(xyz, new_xyz, features, W1, b1, g1, bt1, W2, b2, g2, bt2):
    xyzT = jnp.transpose(xyz, (0, 2, 1))
    j = np.arange(_N)
    pk = jnp.asarray(
        ((j[:, None] // 16 == np.arange(_NWORD)[None, :])
         * (1 << (j % 16))[:, None]).astype(np.float32), jnp.bfloat16)
    words, table = _pack_words(xyzT, new_xyz, pk, features)
    xg = _select_gather(words.reshape(_B * _M * _NWORD),
                        table, jnp.asarray(_PCB_NP),
                        jnp.asarray(_BSL_NP))

    nq_flat = new_xyz.reshape(_B * _M, 3)
    w1p = jnp.pad(W1, ((0, 0), (0, _P - 3 - _C))).T
    w1x = W1[:, :3].T

    nblk = _BT // _PB
    y1, s1, ss1 = pl.pallas_call(
        _k1_body,
        grid=(nblk,),
        in_specs=[
            pl.BlockSpec((_PB, _P), lambda i: (i, 0)),
            pl.BlockSpec((_QB, 3), lambda i: (i, 0)),
            _whole((_P, _H1)),
            _whole((3, _H1)),
            _whole((1, _H1)),
        ],
        out_specs=[
            pl.BlockSpec((_PB, _H1), lambda i: (i, 0)),
            _whole((1, _H1)),
            _whole((1, _H1)),
        ],
        out_shape=[
            jax.ShapeDtypeStruct((_BT, _H1), jnp.float32),
            jax.ShapeDtypeStruct((1, _H1), jnp.float32),
            jax.ShapeDtypeStruct((1, _H1), jnp.float32),
        ],
    )(xg, nq_flat, w1p, w1x, b1[None])

    y2, s2, ss2 = pl.pallas_call(
        _k2_body,
        grid=(nblk,),
        in_specs=[
            pl.BlockSpec((_PB, _H1), lambda i: (i, 0)),
            _whole((1, _H1)),
            _whole((1, _H1)),
            _whole((1, _H1)),
            _whole((1, _H1)),
            _whole((_H1, _H2)),
            _whole((1, _H2)),
        ],
        out_specs=[
            pl.BlockSpec((_PB, _H2), lambda i: (i, 0)),
            _whole((1, _H2)),
            _whole((1, _H2)),
        ],
        out_shape=[
            jax.ShapeDtypeStruct((_BT, _H2), jnp.float32),
            jax.ShapeDtypeStruct((1, _H2), jnp.float32),
            jax.ShapeDtypeStruct((1, _H2), jnp.float32),
        ],
    )(y1, s1, ss1, g1[None], bt1[None], W2.T, b2[None])

    out = pl.pallas_call(
        _k3_body,
        grid=(_B * _M // _QB,),
        in_specs=[
            pl.BlockSpec((_QB, _NS, _H2), lambda i: (i, 0, 0)),
            _whole((1, _H2)),
            _whole((1, _H2)),
            _whole((1, _H2)),
            _whole((1, _H2)),
        ],
        out_specs=pl.BlockSpec((1, _H2, _QB),
                               lambda i: (i // (_M // _QB), 0, i % (_M // _QB))),
        out_shape=jax.ShapeDtypeStruct((_B, _H2, _M), jnp.float32),
    )(y2.reshape(_B * _M, _NS, _H2), s2, ss2, g2[None], bt2[None])

    return (new_xyz, out)
